# baseline (speedup 1.0000x reference)
import functools

import jax
import jax.numpy as jnp
from jax import lax
from jax.experimental import pallas as pl
from jax.experimental.pallas import tpu as pltpu
from jax.experimental.pallas import tpu_sc as plsc

_EPS = 1e-16
_F32 = jnp.float32


def _leaky(x):
    return jnp.maximum(x, 0.2 * x)


def _dgT(a, b):
    return lax.dot_general(a, b, (((0,), (0,)), ((), ())),
                           preferred_element_type=_F32)


def _dot(a, b):
    return jnp.dot(a, b, preferred_element_type=_F32)


def _cluster_body(*refs):
    idx1_ref, idx3_ref = refs[0], refs[1]
    o1_ref, o3_ref, l2_ref, r2_ref, par_ref = refs[-5:]
    per = refs[2:-5]
    s2 = (lax.broadcasted_iota(jnp.int32, (20, 2), 0) // 10 ==
          lax.broadcasted_iota(jnp.int32, (20, 2), 1)).astype(_F32)
    s2e = (lax.broadcasted_iota(jnp.int32, (2, 20), 1) // 10 ==
           lax.broadcasted_iota(jnp.int32, (2, 20), 0)).astype(_F32)
    s8e = (lax.broadcasted_iota(jnp.int32, (8, 128), 1) // 16 ==
           lax.broadcasted_iota(jnp.int32, (8, 128), 0)).astype(_F32)
    n1 = lax.broadcasted_iota(jnp.int32, (85, 1448), 0)
    n3 = lax.broadcasted_iota(jnp.int32, (85, 1368), 0)

    for c in range(4):
        (x1_ref, x3_ref, x2_ref, wl1_ref, wr1_ref, attc_ref, attr_ref,
         b1_ref, pw_ref, pb_ref, ws_ref, bs_ref, wd_ref, bd_ref, lw_ref,
         lb_ref, wl2_ref, wr2_ref, att2_ref) = per[c * 19:(c + 1) * 19]

        x1 = x1_ref[...]
        xl = _dot(x1, wl1_ref[...])
        xr = _dot(x1, wr1_ref[...])
        att2d = s2 * attc_ref[...]
        hi = xl.max(0, keepdims=True) + xr.max(0, keepdims=True)
        lo = xl.min(0, keepdims=True) + xr.min(0, keepdims=True)
        af = attr_ref[...]
        bnd = jnp.maximum(af * _leaky(hi), af * _leaky(lo))
        b_head = _dot(bnd, s2)

        o_st = (n1 == idx1_ref[c, 0:1, :]).astype(_F32)
        o_dt = (n1 == idx1_ref[c, 1:2, :]).astype(_F32)
        xj = _dgT(o_st, xl)
        xi = _dgT(o_dt, xr)
        e1 = _leaky(xi + xj)
        t1 = jnp.exp(_dot(e1, att2d) - b_head)
        t1e = _dot(t1, s2e)
        num1 = _dot(o_dt, xj * t1e)
        den1 = _dot(_dot(o_dt, t1), s2e)
        o1_ref[c] = num1 / (den1 + _EPS) + b1_ref[...]

        x3 = x3_ref[...]
        h3 = _dot(x3, pw_ref[...]) + pb_ref[...]
        a_s = _dot(x3, ws_ref[...]) + bs_ref[...]
        a_d = _dot(x3, wd_ref[...]) + bd_ref[...]
        b3 = _leaky(a_s.max(0, keepdims=True) + a_d.max(0, keepdims=True))
        p_st = (n3 == idx3_ref[c, 0:1, :]).astype(_F32)
        p_dt = (n3 == idx3_ref[c, 1:2, :]).astype(_F32)
        lg3 = _leaky(_dgT(p_st, a_s) + _dgT(p_dt, a_d)) - b3
        t3 = jnp.exp(lg3)
        xj3 = _dgT(p_st, h3)
        t3e = _dot(t3, s8e)
        num3 = _dot(p_dt, xj3 * t3e)
        den3 = _dot(_dot(p_dt, t3), s8e)
        o3p = jax.nn.relu(num3 / (den3 + _EPS))
        o3_ref[c] = _dot(o3p, lw_ref[...]) + lb_ref[...]

        x2 = x2_ref[...]
        l2 = _dot(x2, wl2_ref[...])
        r2 = _dot(x2, wr2_ref[...])
        l2_ref[c] = l2
        r2_ref[c] = r2
        att2 = att2_ref[...]
        hi2 = l2.max(0, keepdims=True) + r2.max(0, keepdims=True)
        lo2 = l2.min(0, keepdims=True) + r2.min(0, keepdims=True)
        b2 = jnp.maximum(att2 * _leaky(hi2), att2 * _leaky(lo2))
        par_ref[c] = jnp.concatenate(
            [jnp.broadcast_to(att2, (1, 16)),
             jnp.broadcast_to(b2, (1, 16))], axis=0)


def _sc_edge_body(l_hbm, r_hbm, e_hbm, par_hbm, out_hbm,
                  l_tbl, r_tbl, num_tbl, den_tbl, src_v, dst_v, par_v, sem):
    wid = lax.axis_index("s") * 2 + lax.axis_index("c")
    cl = wid // 8
    ch = lax.rem(wid, 8)
    hs = [
        pltpu.async_copy(l_hbm.at[pl.ds(cl * 2400, 2400)], l_tbl, sem),
        pltpu.async_copy(r_hbm.at[pl.ds(cl * 2400, 2400)], r_tbl, sem),
        pltpu.async_copy(e_hbm.at[pl.ds(cl * 76800 + ch * 4800, 4800)],
                         src_v, sem),
        pltpu.async_copy(
            e_hbm.at[pl.ds(cl * 76800 + 38400 + ch * 4800, 4800)],
            dst_v, sem),
        pltpu.async_copy(par_hbm.at[pl.ds(cl * 32, 32)], par_v, sem),
    ]
    for h in hs:
        h.wait()
    att = par_v[pl.ds(0, 16)]
    bnd = par_v[pl.ds(16, 16)]

    @pl.when(ch == 0)
    def _init_self():
        def body(i, carry):
            for j in range(2):
                b = i * 32 + j * 16
                l16 = l_tbl[pl.ds(b, 16)]
                r16 = r_tbl[pl.ds(b, 16)]
                t = jnp.exp(att * _leaky(l16 + r16) - bnd)
                den_tbl[pl.ds(b, 16)] = t
                num_tbl[pl.ds(b, 16)] = t * l16
            return carry
        lax.fori_loop(0, 1, body, 0)

    @pl.when(ch != 0)
    def _init_zero():
        def body(i, carry):
            z = jnp.zeros((16,), _F32)
            for j in range(2):
                b = i * 32 + j * 16
                den_tbl[pl.ds(b, 16)] = z
                num_tbl[pl.ds(b, 16)] = z
            return carry
        lax.fori_loop(0, 1, body, 0)

    def body(i, carry):
        for j in range(4):
            b = i * 64 + j * 16
            s16 = src_v[pl.ds(b, 16)]
            d16 = dst_v[pl.ds(b, 16)]
            l16 = plsc.load_gather(l_tbl, [s16])
            r16 = plsc.load_gather(r_tbl, [d16])
            t = jnp.exp(att * _leaky(l16 + r16) - bnd)
            plsc.addupdate_scatter(den_tbl, [d16], t)
            plsc.addupdate_scatter(num_tbl, [d16], t * l16)
        return carry
    lax.fori_loop(0, 1, body, 0)

    pltpu.sync_copy(num_tbl, out_hbm.at[pl.ds(wid * 4800, 2400)])
    pltpu.sync_copy(den_tbl, out_hbm.at[pl.ds(wid * 4800 + 2400, 2400)])


def _run_sc_stage(l2f, r2f, edges, parf):
    mesh = plsc.VectorSubcoreMesh(core_axis_name="c", subcore_axis_name="s")
    k = functools.partial(
        pl.kernel,
        out_type=jax.ShapeDtypeStruct((153600,), _F32),
        mesh=mesh,
        scratch_types=[
            pltpu.VMEM((2400,), _F32),
            pltpu.VMEM((2400,), _F32),
            pltpu.VMEM((2400,), _F32),
            pltpu.VMEM((2400,), _F32),
            pltpu.VMEM((4800,), jnp.int32),
            pltpu.VMEM((4800,), jnp.int32),
            pltpu.VMEM((32,), _F32),
            pltpu.SemaphoreType.DMA,
        ],
        compiler_params=pltpu.CompilerParams(needs_layout_passes=False),
    )(_sc_edge_body)
    return k(l2f, r2f, edges, parf)


def _final_body(*refs):
    (o1_ref, o3_ref, nd_ref, t_ref, tw_ref, tb_ref, sfw_ref, sfb_ref,
     f1w_ref, f1b_ref, f2w_ref, f2b_ref) = refs[:12]
    o_ref = refs[-1]
    per = refs[12:-1]
    s = jnp.zeros((100, 1), _F32)
    for c in range(4):
        b2_ref, fw_ref, fb_ref = per[c * 3:(c + 1) * 3]
        nacc = nd_ref[c, 0, 0]
        dacc = nd_ref[c, 0, 1]
        for k in range(1, 8):
            nacc = nacc + nd_ref[c, k, 0]
            dacc = dacc + nd_ref[c, k, 1]
        o2 = nacc / (dacc + _EPS) + b2_ref[...]
        xc = (_dot(jax.nn.relu(o1_ref[c]), fw_ref[0:17, :]) +
              _dot(jax.nn.relu(o2), fw_ref[17:41, :]) +
              _dot(jax.nn.relu(o3_ref[c]), fw_ref[41:58, :]) +
              fb_ref[...])
        s = s + _dot(jax.nn.relu(xc), sfw_ref[7 * c:7 * c + 7, :])
    s = s + sfb_ref[...]
    t14 = jax.nn.relu(_dot(t_ref[...], tw_ref[...]) + tb_ref[...])
    y0 = _dgT(s[0:50, :], f1w_ref[0:50, :]) + _dot(t14[:, 0:7],
                                                   f1w_ref[50:57, :])
    y1 = _dgT(s[50:100, :], f1w_ref[0:50, :]) + _dot(t14[:, 7:14],
                                                     f1w_ref[50:57, :])
    y0 = y0 + f1b_ref[...]
    y1 = y1 + f1b_ref[...]
    o_ref[...] = (y0 * f2w_ref[0:1, :] + y1 * f2w_ref[1:2, :] +
                  f2b_ref[...])


def kernel(x1_1, x1_2, x1_3, x2_1, x2_2, x2_3, x3_1, x3_2, x3_3, x4_1, x4_2, x4_3, t_data, params, ei1_1, ei1_2, ei1_3, ei2_1, ei2_2, ei2_3, ei3_1, ei3_2, ei3_3, ei4_1, ei4_2, ei4_3):
    i32 = jnp.int32
    xs1 = [x1_1, x2_1, x3_1, x4_1]
    xs2 = [x1_2, x2_2, x3_2, x4_2]
    xs3 = [x1_3, x2_3, x3_3, x4_3]
    e1 = [ei1_1, ei2_1, ei3_1, ei4_1]
    e2 = [ei1_2, ei2_2, ei3_2, ei4_2]
    e3 = [ei1_3, ei2_3, ei3_3, ei4_3]
    ps = [params['c%d' % (c + 1)] for c in range(4)]

    loops = jnp.broadcast_to(jnp.arange(85, dtype=i32), (4, 2, 85))
    pad3 = jnp.full((4, 2, 3), 127, i32)
    pad8 = jnp.full((4, 2, 8), 127, i32)
    idx1 = jnp.concatenate([jnp.stack(e1), loops, pad3], axis=2)
    idx3 = jnp.concatenate([jnp.stack(e3), pad8], axis=2)

    s8 = (jnp.arange(128)[:, None] // 16 ==
          jnp.arange(8)[None, :]).astype(_F32)
    per_in = []
    for c in range(4):
        p = ps[c]
        h = p['han']
        asf = h['att_src'].reshape(1, 128)
        adf = h['att_dst'].reshape(1, 128)
        per_in += [
            xs1[c], xs3[c], xs2[c],
            p['gat1']['Wl'], p['gat1']['Wr'],
            p['gat1']['att'].reshape(20, 1), p['gat1']['att'].reshape(1, 20),
            p['gat1']['b'].reshape(1, 20),
            h['proj_W'], h['proj_b'].reshape(1, 128),
            (h['proj_W'] * asf) @ s8, (h['proj_b'].reshape(1, 128) * asf) @ s8,
            (h['proj_W'] * adf) @ s8, (h['proj_b'].reshape(1, 128) * adf) @ s8,
            h['lin_W'], h['lin_b'].reshape(1, 20),
            p['gat2']['Wl'], p['gat2']['Wr'], p['gat2']['att'].reshape(1, 1),
        ]

    o1, o3, l2, r2, par = pl.pallas_call(
        _cluster_body,
        out_shape=[jax.ShapeDtypeStruct((4, 85, 20), _F32),
                   jax.ShapeDtypeStruct((4, 85, 20), _F32),
                   jax.ShapeDtypeStruct((4, 2400, 1), _F32),
                   jax.ShapeDtypeStruct((4, 2400, 1), _F32),
                   jax.ShapeDtypeStruct((4, 2, 16), _F32)],
    )(idx1, idx3, *per_in)

    edges = jnp.stack(e2).reshape(-1)
    ndp = _run_sc_stage(l2.reshape(-1), r2.reshape(-1), edges,
                        par.reshape(-1))
    nd = ndp.reshape(4, 8, 2, 100, 24)

    final_in = [o1.reshape(4, 100, 17), o3.reshape(4, 100, 17), nd,
                t_data, params['t_W'], params['t_b'].reshape(1, 14),
                params['sfcn_W'], params['sfcn_b'].reshape(1, 1),
                params['fcn1_W'], params['fcn1_b'].reshape(1, 1),
                params['fcn2_W'], params['fcn2_b'].reshape(1, 1)]
    for c in range(4):
        final_in += [ps[c]['gat2']['b'].reshape(1, 1), ps[c]['fcn_W'],
                     ps[c]['fcn_b'].reshape(1, 7)]
    out = pl.pallas_call(
        _final_body,
        out_shape=jax.ShapeDtypeStruct((1, 1), _F32),
    )(*final_in)
    return out.reshape(1)


# speedup vs baseline: 1.0028x; 1.0028x over previous
import functools

import jax
import jax.numpy as jnp
from jax import lax
from jax.experimental import pallas as pl
from jax.experimental.pallas import tpu as pltpu
from jax.experimental.pallas import tpu_sc as plsc

_EPS = 1e-16
_F32 = jnp.float32


def _leaky(x):
    return jnp.maximum(x, 0.2 * x)


def _dgT(a, b):
    return lax.dot_general(a, b, (((0,), (0,)), ((), ())),
                           preferred_element_type=_F32)


def _dot(a, b):
    return jnp.dot(a, b, preferred_element_type=_F32)


def _cluster_body(*refs):
    idx1_ref, idx3_ref = refs[0], refs[1]
    o1_ref, o3_ref, l2_ref, r2_ref, par_ref = refs[-5:]
    per = refs[2:-5]
    s2 = (lax.broadcasted_iota(jnp.int32, (20, 2), 0) // 10 ==
          lax.broadcasted_iota(jnp.int32, (20, 2), 1)).astype(_F32)
    s2e = (lax.broadcasted_iota(jnp.int32, (2, 20), 1) // 10 ==
           lax.broadcasted_iota(jnp.int32, (2, 20), 0)).astype(_F32)
    s8e = (lax.broadcasted_iota(jnp.int32, (8, 128), 1) // 16 ==
           lax.broadcasted_iota(jnp.int32, (8, 128), 0)).astype(_F32)
    n1 = lax.broadcasted_iota(jnp.int32, (85, 1448), 0)
    n3 = lax.broadcasted_iota(jnp.int32, (85, 1368), 0)

    for c in range(4):
        (x1_ref, x3_ref, x2_ref, wl1_ref, wr1_ref, attc_ref, attr_ref,
         b1_ref, pw_ref, pb_ref, ws_ref, bs_ref, wd_ref, bd_ref, lw_ref,
         lb_ref, wl2_ref, wr2_ref, att2_ref) = per[c * 19:(c + 1) * 19]

        x1 = x1_ref[...]
        xl = _dot(x1, wl1_ref[...])
        xr = _dot(x1, wr1_ref[...])
        att2d = s2 * attc_ref[...]
        hi = xl.max(0, keepdims=True) + xr.max(0, keepdims=True)
        lo = xl.min(0, keepdims=True) + xr.min(0, keepdims=True)
        af = attr_ref[...]
        bnd = jnp.maximum(af * _leaky(hi), af * _leaky(lo))
        b_head = _dot(bnd, s2)

        o_st = (n1 == idx1_ref[c, 0:1, :]).astype(_F32)
        o_dt = (n1 == idx1_ref[c, 1:2, :]).astype(_F32)
        xj = _dgT(o_st, xl)
        xi = _dgT(o_dt, xr)
        e1 = _leaky(xi + xj)
        t1 = jnp.exp(_dot(e1, att2d) - b_head)
        t1e = _dot(t1, s2e)
        num1 = _dot(o_dt, xj * t1e)
        den1 = _dot(_dot(o_dt, t1), s2e)
        o1_ref[c] = num1 / (den1 + _EPS) + b1_ref[...]

        x3 = x3_ref[...]
        h3 = _dot(x3, pw_ref[...]) + pb_ref[...]
        a_s = _dot(x3, ws_ref[...]) + bs_ref[...]
        a_d = _dot(x3, wd_ref[...]) + bd_ref[...]
        b3 = _leaky(a_s.max(0, keepdims=True) + a_d.max(0, keepdims=True))
        p_st = (n3 == idx3_ref[c, 0:1, :]).astype(_F32)
        p_dt = (n3 == idx3_ref[c, 1:2, :]).astype(_F32)
        lg3 = _leaky(_dgT(p_st, a_s) + _dgT(p_dt, a_d)) - b3
        t3 = jnp.exp(lg3)
        xj3 = _dgT(p_st, h3)
        t3e = _dot(t3, s8e)
        num3 = _dot(p_dt, xj3 * t3e)
        den3 = _dot(_dot(p_dt, t3), s8e)
        o3p = jax.nn.relu(num3 / (den3 + _EPS))
        o3_ref[c] = _dot(o3p, lw_ref[...]) + lb_ref[...]

        x2 = x2_ref[...]
        l2 = _dot(x2, wl2_ref[...])
        r2 = _dot(x2, wr2_ref[...])
        l2_ref[c] = l2
        r2_ref[c] = r2
        att2 = att2_ref[...]
        hi2 = l2.max(0, keepdims=True) + r2.max(0, keepdims=True)
        lo2 = l2.min(0, keepdims=True) + r2.min(0, keepdims=True)
        b2 = jnp.maximum(att2 * _leaky(hi2), att2 * _leaky(lo2))
        par_ref[c] = jnp.concatenate(
            [jnp.broadcast_to(att2, (1, 16)),
             jnp.broadcast_to(b2, (1, 16))], axis=0)


def _sc_edge_body(l_hbm, r_hbm, e_hbm, par_hbm, out_hbm,
                  l_tbl, r_tbl, num_tbl, den_tbl, src_v, dst_v, par_v, sem):
    wid = lax.axis_index("s") * 2 + lax.axis_index("c")
    cl = wid // 8
    pltpu.async_copy(par_hbm.at[pl.ds(cl * 32, 32)], par_v, sem).wait()
    pltpu.sync_copy(par_v, out_hbm.at[pl.ds(wid * 4800, 32)])


def _run_sc_stage(l2f, r2f, edges, parf):
    mesh = plsc.VectorSubcoreMesh(core_axis_name="c", subcore_axis_name="s")
    k = functools.partial(
        pl.kernel,
        out_type=jax.ShapeDtypeStruct((153600,), _F32),
        mesh=mesh,
        scratch_types=[
            pltpu.VMEM((2400,), _F32),
            pltpu.VMEM((2400,), _F32),
            pltpu.VMEM((2400,), _F32),
            pltpu.VMEM((2400,), _F32),
            pltpu.VMEM((4800,), jnp.int32),
            pltpu.VMEM((4800,), jnp.int32),
            pltpu.VMEM((32,), _F32),
            pltpu.SemaphoreType.DMA,
        ],
        compiler_params=pltpu.CompilerParams(needs_layout_passes=False),
    )(_sc_edge_body)
    return k(l2f, r2f, edges, parf)


def _final_body(*refs):
    (o1_ref, o3_ref, nd_ref, t_ref, tw_ref, tb_ref, sfw_ref, sfb_ref,
     f1w_ref, f1b_ref, f2w_ref, f2b_ref) = refs[:12]
    o_ref = refs[-1]
    per = refs[12:-1]
    s = jnp.zeros((100, 1), _F32)
    for c in range(4):
        b2_ref, fw_ref, fb_ref = per[c * 3:(c + 1) * 3]
        nacc = nd_ref[c, 0, 0]
        dacc = nd_ref[c, 0, 1]
        for k in range(1, 8):
            nacc = nacc + nd_ref[c, k, 0]
            dacc = dacc + nd_ref[c, k, 1]
        o2 = nacc / (dacc + _EPS) + b2_ref[...]
        xc = (_dot(jax.nn.relu(o1_ref[c]), fw_ref[0:17, :]) +
              _dot(jax.nn.relu(o2), fw_ref[17:41, :]) +
              _dot(jax.nn.relu(o3_ref[c]), fw_ref[41:58, :]) +
              fb_ref[...])
        s = s + _dot(jax.nn.relu(xc), sfw_ref[7 * c:7 * c + 7, :])
    s = s + sfb_ref[...]
    t14 = jax.nn.relu(_dot(t_ref[...], tw_ref[...]) + tb_ref[...])
    y0 = _dgT(s[0:50, :], f1w_ref[0:50, :]) + _dot(t14[:, 0:7],
                                                   f1w_ref[50:57, :])
    y1 = _dgT(s[50:100, :], f1w_ref[0:50, :]) + _dot(t14[:, 7:14],
                                                     f1w_ref[50:57, :])
    y0 = y0 + f1b_ref[...]
    y1 = y1 + f1b_ref[...]
    o_ref[...] = (y0 * f2w_ref[0:1, :] + y1 * f2w_ref[1:2, :] +
                  f2b_ref[...])


def kernel(x1_1, x1_2, x1_3, x2_1, x2_2, x2_3, x3_1, x3_2, x3_3, x4_1, x4_2, x4_3, t_data, params, ei1_1, ei1_2, ei1_3, ei2_1, ei2_2, ei2_3, ei3_1, ei3_2, ei3_3, ei4_1, ei4_2, ei4_3):
    i32 = jnp.int32
    xs1 = [x1_1, x2_1, x3_1, x4_1]
    xs2 = [x1_2, x2_2, x3_2, x4_2]
    xs3 = [x1_3, x2_3, x3_3, x4_3]
    e1 = [ei1_1, ei2_1, ei3_1, ei4_1]
    e2 = [ei1_2, ei2_2, ei3_2, ei4_2]
    e3 = [ei1_3, ei2_3, ei3_3, ei4_3]
    ps = [params['c%d' % (c + 1)] for c in range(4)]

    loops = jnp.broadcast_to(jnp.arange(85, dtype=i32), (4, 2, 85))
    pad3 = jnp.full((4, 2, 3), 127, i32)
    pad8 = jnp.full((4, 2, 8), 127, i32)
    idx1 = jnp.concatenate([jnp.stack(e1), loops, pad3], axis=2)
    idx3 = jnp.concatenate([jnp.stack(e3), pad8], axis=2)

    s8 = (jnp.arange(128)[:, None] // 16 ==
          jnp.arange(8)[None, :]).astype(_F32)
    per_in = []
    for c in range(4):
        p = ps[c]
        h = p['han']
        asf = h['att_src'].reshape(1, 128)
        adf = h['att_dst'].reshape(1, 128)
        per_in += [
            xs1[c], xs3[c], xs2[c],
            p['gat1']['Wl'], p['gat1']['Wr'],
            p['gat1']['att'].reshape(20, 1), p['gat1']['att'].reshape(1, 20),
            p['gat1']['b'].reshape(1, 20),
            h['proj_W'], h['proj_b'].reshape(1, 128),
            (h['proj_W'] * asf) @ s8, (h['proj_b'].reshape(1, 128) * asf) @ s8,
            (h['proj_W'] * adf) @ s8, (h['proj_b'].reshape(1, 128) * adf) @ s8,
            h['lin_W'], h['lin_b'].reshape(1, 20),
            p['gat2']['Wl'], p['gat2']['Wr'], p['gat2']['att'].reshape(1, 1),
        ]

    o1, o3, l2, r2, par = pl.pallas_call(
        _cluster_body,
        out_shape=[jax.ShapeDtypeStruct((4, 85, 20), _F32),
                   jax.ShapeDtypeStruct((4, 85, 20), _F32),
                   jax.ShapeDtypeStruct((4, 2400, 1), _F32),
                   jax.ShapeDtypeStruct((4, 2400, 1), _F32),
                   jax.ShapeDtypeStruct((4, 2, 16), _F32)],
    )(idx1, idx3, *per_in)

    edges = jnp.stack(e2).reshape(-1)
    ndp = _run_sc_stage(l2.reshape(-1), r2.reshape(-1), edges,
                        par.reshape(-1))
    nd = ndp.reshape(4, 8, 2, 100, 24)

    final_in = [o1.reshape(4, 100, 17), o3.reshape(4, 100, 17), nd,
                t_data, params['t_W'], params['t_b'].reshape(1, 14),
                params['sfcn_W'], params['sfcn_b'].reshape(1, 1),
                params['fcn1_W'], params['fcn1_b'].reshape(1, 1),
                params['fcn2_W'], params['fcn2_b'].reshape(1, 1)]
    for c in range(4):
        final_in += [ps[c]['gat2']['b'].reshape(1, 1), ps[c]['fcn_W'],
                     ps[c]['fcn_b'].reshape(1, 7)]
    out = pl.pallas_call(
        _final_body,
        out_shape=jax.ShapeDtypeStruct((1, 1), _F32),
    )(*final_in)
    return out.reshape(1)


# speedup vs baseline: 1.0902x; 1.0871x over previous
import functools

import jax
import jax.numpy as jnp
from jax import lax
from jax.experimental import pallas as pl
from jax.experimental.pallas import tpu as pltpu
from jax.experimental.pallas import tpu_sc as plsc

_EPS = 1e-16
_F32 = jnp.float32


def _leaky(x):
    return jnp.maximum(x, 0.2 * x)


def _dgT(a, b):
    return lax.dot_general(a, b, (((0,), (0,)), ((), ())),
                           preferred_element_type=_F32)


def _dot(a, b):
    return jnp.dot(a, b, preferred_element_type=_F32)


def _prep_body(*refs):
    l2_ref, r2_ref, par_ref = refs[-3:]
    per = refs[:-3]
    for c in range(4):
        x2_ref, wl2_ref, wr2_ref, att2_ref = per[c * 4:(c + 1) * 4]
        x2 = x2_ref[...]
        l2 = _dot(x2, wl2_ref[...])
        r2 = _dot(x2, wr2_ref[...])
        l2_ref[c] = l2
        r2_ref[c] = r2
        att2 = att2_ref[...]
        hi2 = l2.max(0, keepdims=True) + r2.max(0, keepdims=True)
        lo2 = l2.min(0, keepdims=True) + r2.min(0, keepdims=True)
        b2 = jnp.maximum(att2 * _leaky(hi2), att2 * _leaky(lo2))
        par_ref[c] = jnp.concatenate(
            [jnp.broadcast_to(att2, (1, 16)),
             jnp.broadcast_to(b2, (1, 16))], axis=0)


def _cluster_body(*refs):
    idx1_ref, idx3_ref = refs[0], refs[1]
    o1_ref, o3_ref = refs[-2:]
    per = refs[2:-2]
    s2 = (lax.broadcasted_iota(jnp.int32, (20, 2), 0) // 10 ==
          lax.broadcasted_iota(jnp.int32, (20, 2), 1)).astype(_F32)
    s2e = (lax.broadcasted_iota(jnp.int32, (2, 20), 1) // 10 ==
           lax.broadcasted_iota(jnp.int32, (2, 20), 0)).astype(_F32)
    s8e = (lax.broadcasted_iota(jnp.int32, (8, 128), 1) // 16 ==
           lax.broadcasted_iota(jnp.int32, (8, 128), 0)).astype(_F32)
    n1 = lax.broadcasted_iota(jnp.int32, (85, 1448), 0)
    n3 = lax.broadcasted_iota(jnp.int32, (85, 1368), 0)

    for c in range(4):
        (x1_ref, x3_ref, wl1_ref, wr1_ref, attc_ref, attr_ref,
         b1_ref, pw_ref, pb_ref, ws_ref, bs_ref, wd_ref, bd_ref, lw_ref,
         lb_ref) = per[c * 15:(c + 1) * 15]

        x1 = x1_ref[...]
        xl = _dot(x1, wl1_ref[...])
        xr = _dot(x1, wr1_ref[...])
        att2d = s2 * attc_ref[...]
        hi = xl.max(0, keepdims=True) + xr.max(0, keepdims=True)
        lo = xl.min(0, keepdims=True) + xr.min(0, keepdims=True)
        af = attr_ref[...]
        bnd = jnp.maximum(af * _leaky(hi), af * _leaky(lo))
        b_head = _dot(bnd, s2)

        o_st = (n1 == idx1_ref[c, 0:1, :]).astype(_F32)
        o_dt = (n1 == idx1_ref[c, 1:2, :]).astype(_F32)
        xj = _dgT(o_st, xl)
        xi = _dgT(o_dt, xr)
        e1 = _leaky(xi + xj)
        t1 = jnp.exp(_dot(e1, att2d) - b_head)
        t1e = _dot(t1, s2e)
        num1 = _dot(o_dt, xj * t1e)
        den1 = _dot(_dot(o_dt, t1), s2e)
        o1_ref[c] = num1 / (den1 + _EPS) + b1_ref[...]

        x3 = x3_ref[...]
        h3 = _dot(x3, pw_ref[...]) + pb_ref[...]
        a_s = _dot(x3, ws_ref[...]) + bs_ref[...]
        a_d = _dot(x3, wd_ref[...]) + bd_ref[...]
        b3 = _leaky(a_s.max(0, keepdims=True) + a_d.max(0, keepdims=True))
        p_st = (n3 == idx3_ref[c, 0:1, :]).astype(_F32)
        p_dt = (n3 == idx3_ref[c, 1:2, :]).astype(_F32)
        lg3 = _leaky(_dgT(p_st, a_s) + _dgT(p_dt, a_d)) - b3
        t3 = jnp.exp(lg3)
        xj3 = _dgT(p_st, h3)
        t3e = _dot(t3, s8e)
        num3 = _dot(p_dt, xj3 * t3e)
        den3 = _dot(_dot(p_dt, t3), s8e)
        o3p = jax.nn.relu(num3 / (den3 + _EPS))
        o3_ref[c] = _dot(o3p, lw_ref[...]) + lb_ref[...]


def _sc_edge_body(l_hbm, r_hbm, e_hbm, par_hbm, out_hbm,
                  l_tbl, r_tbl, num_tbl, den_tbl, src_v, dst_v, par_v, sem):
    wid = lax.axis_index("s") * 2 + lax.axis_index("c")
    cl = wid // 8
    ch = lax.rem(wid, 8)
    hs = [
        pltpu.async_copy(l_hbm.at[pl.ds(cl * 2400, 2400)], l_tbl, sem),
        pltpu.async_copy(r_hbm.at[pl.ds(cl * 2400, 2400)], r_tbl, sem),
        pltpu.async_copy(e_hbm.at[pl.ds(cl * 76800 + ch * 4800, 4800)],
                         src_v, sem),
        pltpu.async_copy(
            e_hbm.at[pl.ds(cl * 76800 + 38400 + ch * 4800, 4800)],
            dst_v, sem),
        pltpu.async_copy(par_hbm.at[pl.ds(cl * 32, 32)], par_v, sem),
    ]
    for h in hs:
        h.wait()
    att = par_v[pl.ds(0, 16)]
    bnd = par_v[pl.ds(16, 16)]

    @pl.when(ch == 0)
    def _init_self():
        def body(i, carry):
            for j in range(2):
                b = i * 32 + j * 16
                l16 = l_tbl[pl.ds(b, 16)]
                r16 = r_tbl[pl.ds(b, 16)]
                t = jnp.exp(att * _leaky(l16 + r16) - bnd)
                den_tbl[pl.ds(b, 16)] = t
                num_tbl[pl.ds(b, 16)] = t * l16
            return carry
        lax.fori_loop(0, 75, body, 0)

    @pl.when(ch != 0)
    def _init_zero():
        def body(i, carry):
            z = jnp.zeros((16,), _F32)
            for j in range(2):
                b = i * 32 + j * 16
                den_tbl[pl.ds(b, 16)] = z
                num_tbl[pl.ds(b, 16)] = z
            return carry
        lax.fori_loop(0, 75, body, 0)

    def body(i, carry):
        for j in range(4):
            b = i * 64 + j * 16
            s16 = src_v[pl.ds(b, 16)]
            d16 = dst_v[pl.ds(b, 16)]
            l16 = plsc.load_gather(l_tbl, [s16])
            r16 = plsc.load_gather(r_tbl, [d16])
            t = jnp.exp(att * _leaky(l16 + r16) - bnd)
            plsc.addupdate_scatter(den_tbl, [d16], t)
            plsc.addupdate_scatter(num_tbl, [d16], t * l16)
        return carry
    lax.fori_loop(0, 75, body, 0)

    pltpu.sync_copy(num_tbl, out_hbm.at[pl.ds(wid * 4800, 2400)])
    pltpu.sync_copy(den_tbl, out_hbm.at[pl.ds(wid * 4800 + 2400, 2400)])


def _run_sc_stage(l2f, r2f, edges, parf):
    mesh = plsc.VectorSubcoreMesh(core_axis_name="c", subcore_axis_name="s")
    k = functools.partial(
        pl.kernel,
        out_type=jax.ShapeDtypeStruct((153600,), _F32),
        mesh=mesh,
        scratch_types=[
            pltpu.VMEM((2400,), _F32),
            pltpu.VMEM((2400,), _F32),
            pltpu.VMEM((2400,), _F32),
            pltpu.VMEM((2400,), _F32),
            pltpu.VMEM((4800,), jnp.int32),
            pltpu.VMEM((4800,), jnp.int32),
            pltpu.VMEM((32,), _F32),
            pltpu.SemaphoreType.DMA,
        ],
        compiler_params=pltpu.CompilerParams(needs_layout_passes=False),
    )(_sc_edge_body)
    return k(l2f, r2f, edges, parf)


def _final_body(*refs):
    (o1_ref, o3_ref, nd_ref, t_ref, tw_ref, tb_ref, sfw_ref, sfb_ref,
     f1w_ref, f1b_ref, f2w_ref, f2b_ref) = refs[:12]
    o_ref = refs[-1]
    per = refs[12:-1]
    s = jnp.zeros((100, 1), _F32)
    for c in range(4):
        b2_ref, fw_ref, fb_ref = per[c * 3:(c + 1) * 3]
        nacc = nd_ref[c, 0, 0]
        dacc = nd_ref[c, 0, 1]
        for k in range(1, 8):
            nacc = nacc + nd_ref[c, k, 0]
            dacc = dacc + nd_ref[c, k, 1]
        o2 = nacc / (dacc + _EPS) + b2_ref[...]
        xc = (_dot(jax.nn.relu(o1_ref[c]), fw_ref[0:17, :]) +
              _dot(jax.nn.relu(o2), fw_ref[17:41, :]) +
              _dot(jax.nn.relu(o3_ref[c]), fw_ref[41:58, :]) +
              fb_ref[...])
        s = s + _dot(jax.nn.relu(xc), sfw_ref[7 * c:7 * c + 7, :])
    s = s + sfb_ref[...]
    t14 = jax.nn.relu(_dot(t_ref[...], tw_ref[...]) + tb_ref[...])
    y0 = _dgT(s[0:50, :], f1w_ref[0:50, :]) + _dot(t14[:, 0:7],
                                                   f1w_ref[50:57, :])
    y1 = _dgT(s[50:100, :], f1w_ref[0:50, :]) + _dot(t14[:, 7:14],
                                                     f1w_ref[50:57, :])
    y0 = y0 + f1b_ref[...]
    y1 = y1 + f1b_ref[...]
    o_ref[...] = (y0 * f2w_ref[0:1, :] + y1 * f2w_ref[1:2, :] +
                  f2b_ref[...])


def kernel(x1_1, x1_2, x1_3, x2_1, x2_2, x2_3, x3_1, x3_2, x3_3, x4_1, x4_2, x4_3, t_data, params, ei1_1, ei1_2, ei1_3, ei2_1, ei2_2, ei2_3, ei3_1, ei3_2, ei3_3, ei4_1, ei4_2, ei4_3):
    i32 = jnp.int32
    xs1 = [x1_1, x2_1, x3_1, x4_1]
    xs2 = [x1_2, x2_2, x3_2, x4_2]
    xs3 = [x1_3, x2_3, x3_3, x4_3]
    e1 = [ei1_1, ei2_1, ei3_1, ei4_1]
    e2 = [ei1_2, ei2_2, ei3_2, ei4_2]
    e3 = [ei1_3, ei2_3, ei3_3, ei4_3]
    ps = [params['c%d' % (c + 1)] for c in range(4)]

    loops = jnp.broadcast_to(jnp.arange(85, dtype=i32), (4, 2, 85))
    pad3 = jnp.full((4, 2, 3), 127, i32)
    pad8 = jnp.full((4, 2, 8), 127, i32)
    idx1 = jnp.concatenate([jnp.stack(e1), loops, pad3], axis=2)
    idx3 = jnp.concatenate([jnp.stack(e3), pad8], axis=2)

    s8 = (jnp.arange(128)[:, None] // 16 ==
          jnp.arange(8)[None, :]).astype(_F32)
    prep_in = []
    per_in = []
    for c in range(4):
        p = ps[c]
        h = p['han']
        asf = h['att_src'].reshape(1, 128)
        adf = h['att_dst'].reshape(1, 128)
        prep_in += [xs2[c], p['gat2']['Wl'], p['gat2']['Wr'],
                    p['gat2']['att'].reshape(1, 1)]
        per_in += [
            xs1[c], xs3[c],
            p['gat1']['Wl'], p['gat1']['Wr'],
            p['gat1']['att'].reshape(20, 1), p['gat1']['att'].reshape(1, 20),
            p['gat1']['b'].reshape(1, 20),
            h['proj_W'], h['proj_b'].reshape(1, 128),
            (h['proj_W'] * asf) @ s8, (h['proj_b'].reshape(1, 128) * asf) @ s8,
            (h['proj_W'] * adf) @ s8, (h['proj_b'].reshape(1, 128) * adf) @ s8,
            h['lin_W'], h['lin_b'].reshape(1, 20),
        ]

    l2, r2, par = pl.pallas_call(
        _prep_body,
        out_shape=[jax.ShapeDtypeStruct((4, 2400, 1), _F32),
                   jax.ShapeDtypeStruct((4, 2400, 1), _F32),
                   jax.ShapeDtypeStruct((4, 2, 16), _F32)],
    )(*prep_in)

    edges = jnp.stack(e2).reshape(-1)
    ndp = _run_sc_stage(l2.reshape(-1), r2.reshape(-1), edges,
                        par.reshape(-1))

    o1, o3 = pl.pallas_call(
        _cluster_body,
        out_shape=[jax.ShapeDtypeStruct((4, 85, 20), _F32),
                   jax.ShapeDtypeStruct((4, 85, 20), _F32)],
    )(idx1, idx3, *per_in)

    nd = ndp.reshape(4, 8, 2, 100, 24)

    final_in = [o1.reshape(4, 100, 17), o3.reshape(4, 100, 17), nd,
                t_data, params['t_W'], params['t_b'].reshape(1, 14),
                params['sfcn_W'], params['sfcn_b'].reshape(1, 1),
                params['fcn1_W'], params['fcn1_b'].reshape(1, 1),
                params['fcn2_W'], params['fcn2_b'].reshape(1, 1)]
    for c in range(4):
        final_in += [ps[c]['gat2']['b'].reshape(1, 1), ps[c]['fcn_W'],
                     ps[c]['fcn_b'].reshape(1, 7)]
    out = pl.pallas_call(
        _final_body,
        out_shape=jax.ShapeDtypeStruct((1, 1), _F32),
    )(*final_in)
    return out.reshape(1)
